# baseline (speedup 1.0000x reference)
import functools
import math

import jax
import jax.numpy as jnp
from jax.experimental import pallas as pl
from jax.experimental.pallas import tpu as pltpu

N_BLK = 1024


def _node_body(x_ref, a_ref, w_ref, b_ref, o_ref):
    acc = x_ref[...] + a_ref[...]
    y = jnp.dot(acc, w_ref[...], preferred_element_type=jnp.float32)
    o_ref[...] = jnp.maximum(y + b_ref[...], 0.0)


def _node_update(x, aggr, Wf, bf):
    n, din = x.shape
    h = Wf.shape[1]
    grid = (pl.cdiv(n, N_BLK),)
    return pl.pallas_call(
        _node_body,
        grid=grid,
        in_specs=[
            pl.BlockSpec((N_BLK, din), lambda i: (i, 0)),
            pl.BlockSpec((N_BLK, din), lambda i: (i, 0)),
            pl.BlockSpec((din, h), lambda i: (0, 0)),
            pl.BlockSpec((1, h), lambda i: (0, 0)),
        ],
        out_specs=pl.BlockSpec((N_BLK, h), lambda i: (i, 0)),
        out_shape=jax.ShapeDtypeStruct((n, h), jnp.float32),
    )(x, aggr, Wf, bf.reshape(1, h))


def kernel(x, edge_index, edge_attr, batch,
           We1, be1, Wn1, bn1,
           We2, be2, Wn2, bn2,
           We3, be3, Wn3, bn3,
           g1, bt1, g2, bt2, g3, bt3,
           Wl1, bl1, Wl2, bl2):
    G = 64
    n = x.shape[0]
    src = edge_index[0]
    dst = edge_index[1]
    inv = 1.0 / jnp.sqrt(1.0 + 1e-5)

    def fold(Wn, bn, g, bt):
        s = g * inv
        return Wn * s[None, :], bn * s + bt

    Wf1, bf1 = fold(Wn1, bn1, g1, bt1)
    Wf2, bf2 = fold(Wn2, bn2, g2, bt2)
    Wf3, bf3 = fold(Wn3, bn3, g3, bt3)

    h = x
    for We, be, Wf, bf in ((We1, be1, Wf1, bf1),
                           (We2, be2, Wf2, bf2),
                           (We3, be3, Wf3, bf3)):
        e = edge_attr @ We + be
        m = jax.nn.relu(h[src] + e)
        aggr = jax.ops.segment_sum(m, dst, num_segments=n)
        h = _node_update(h, aggr, Wf, bf)

    sums = jax.ops.segment_sum(h, batch, num_segments=G)
    cnts = jax.ops.segment_sum(jnp.ones((n,), dtype=h.dtype), batch,
                               num_segments=G)
    pooled = sums / jnp.maximum(cnts, 1.0)[:, None]
    z = jax.nn.relu(pooled @ Wl1 + bl1)
    z = z @ Wl2 + bl2
    return jax.nn.sigmoid(z)


# speedup vs baseline: 1.2223x; 1.2223x over previous
import functools

import jax
import jax.numpy as jnp
from jax import lax
from jax.experimental import pallas as pl
from jax.experimental.pallas import tpu as pltpu
from jax.experimental.pallas import tpu_sc as plsc

NC = 2
NS = 16
L = 16

HC = 32
BLK = 256
KT = BLK // 128
EB = 2048
NB = 1000


def _sc_message(h_c, e_c, src2, dst2, zrows, n_pad):
    C = h_c.shape[0]
    c_half = C // NC
    e_pad = e_c.shape[1]
    ept = e_pad // NS
    n_blocks = ept // BLK
    rpt = n_pad // NS

    def body(h_hbm, e_hbm, s_hbm, d_hbm, z_hbm, out_hbm,
             sidx, didx, ebuf, gbuf, aggr_sh, sem_in, sem_g):
        core = lax.axis_index("c")
        sub = lax.axis_index("s")

        for cc in range(c_half):
            c = core * c_half + cc
            pltpu.sync_copy(z_hbm, aggr_sh.at[pl.ds(sub * rpt, rpt)])
            plsc.subcore_barrier()

            @pl.loop(0, n_blocks)
            def _blk(g):
                row0 = sub * (ept // 128) + g * KT
                d1 = pltpu.async_copy(s_hbm.at[pl.ds(row0, KT)], sidx, sem_in)
                d2 = pltpu.async_copy(d_hbm.at[pl.ds(row0, KT)], didx, sem_in)
                ebase = sub * ept + g * BLK
                d3 = pltpu.async_copy(e_hbm.at[c].at[pl.ds(ebase, BLK)],
                                      ebuf, sem_in)
                d1.wait()
                d2.wait()
                d3.wait()
                gds = []
                for j in range(KT):
                    gds.append(pltpu.async_copy(
                        h_hbm.at[c].at[sidx.at[j]],
                        gbuf.at[pl.ds(j * 128, 128)], sem_g))
                for gd in gds:
                    gd.wait()

                @pl.loop(0, BLK)
                def _valu(r):
                    for k in range(HC // L):
                        sl = pl.ds(k * L, L)
                        v = gbuf[r, sl] + ebuf[r, sl]
                        gbuf[r, sl] = jnp.maximum(v, 0.0)

                for j in range(KT):
                    pltpu.sync_copy(gbuf.at[pl.ds(j * 128, 128)],
                                    aggr_sh.at[didx.at[j]], add=True)

            plsc.subcore_barrier()
            pltpu.sync_copy(aggr_sh.at[pl.ds(sub * rpt, rpt)],
                            out_hbm.at[c].at[pl.ds(sub * rpt, rpt)])
            plsc.subcore_barrier()

    fn = pl.kernel(
        body,
        out_type=jax.ShapeDtypeStruct((C, n_pad, HC), jnp.float32),
        mesh=plsc.VectorSubcoreMesh(core_axis_name="c", subcore_axis_name="s"),
        compiler_params=pltpu.CompilerParams(use_tc_tiling_on_sc=False),
        scratch_types=[
            pltpu.VMEM((KT, 128), jnp.int32),
            pltpu.VMEM((KT, 128), jnp.int32),
            pltpu.VMEM((BLK, HC), jnp.float32),
            pltpu.VMEM((BLK, HC), jnp.float32),
            pltpu.VMEM_SHARED((n_pad, HC), jnp.float32),
            pltpu.SemaphoreType.DMA,
            pltpu.SemaphoreType.DMA,
        ],
    )
    return fn(h_c, e_c, src2, dst2, zrows)


def _edge_body(ea_ref, w1_ref, b1_ref, w2_ref, b2_ref, w3_ref, b3_ref,
               o1_ref, o2_ref, o3_ref):
    ea = ea_ref[...]
    for o_ref, w_ref, b_ref in ((o1_ref, w1_ref, b1_ref),
                                (o2_ref, w2_ref, b2_ref),
                                (o3_ref, w3_ref, b3_ref)):
        C = o_ref.shape[0]
        for c in range(C):
            y = jnp.dot(ea, w_ref[:, c * HC:(c + 1) * HC],
                        preferred_element_type=jnp.float32)
            o_ref[c] = y + b_ref[0, c * HC:(c + 1) * HC][None, :]


def _edge_lin(ea_p, We1, be1, We2, be2, We3, be3):
    e_pad = ea_p.shape[0]
    grid = (e_pad // EB,)
    d1 = We1.shape[1]
    d23 = We2.shape[1]
    wspec = lambda d: pl.BlockSpec((HC, d), lambda i: (0, 0))
    bspec = lambda d: pl.BlockSpec((1, d), lambda i: (0, 0))
    ospec = lambda C: pl.BlockSpec((C, EB, HC), lambda i: (0, i, 0))
    return pl.pallas_call(
        _edge_body,
        grid=grid,
        in_specs=[pl.BlockSpec((EB, HC), lambda i: (i, 0)),
                  wspec(d1), bspec(d1), wspec(d23), bspec(d23),
                  wspec(d23), bspec(d23)],
        out_specs=[ospec(d1 // HC), ospec(d23 // HC), ospec(d23 // HC)],
        out_shape=[jax.ShapeDtypeStruct((d1 // HC, e_pad, HC), jnp.float32),
                   jax.ShapeDtypeStruct((d23 // HC, e_pad, HC), jnp.float32),
                   jax.ShapeDtypeStruct((d23 // HC, e_pad, HC), jnp.float32)],
    )(ea_p, We1, be1.reshape(1, d1), We2, be2.reshape(1, d23),
      We3, be3.reshape(1, d23))


def _node_body(x_ref, a_ref, w_ref, b_ref, o_ref):
    C_in = x_ref.shape[0]
    acc = b_ref[...]
    for c in range(C_in):
        acc = acc + jnp.dot(x_ref[c] + a_ref[c], w_ref[c],
                            preferred_element_type=jnp.float32)
    y = jnp.maximum(acc, 0.0)
    C_out = o_ref.shape[0]
    for c in range(C_out):
        o_ref[c] = y[:, c * HC:(c + 1) * HC]


def _node_update(x_c, aggr, w_s, b_f):
    C_in, n, _ = x_c.shape
    h = w_s.shape[2]
    C_out = h // HC
    grid = (n // NB,)
    return pl.pallas_call(
        _node_body,
        grid=grid,
        in_specs=[pl.BlockSpec((C_in, NB, HC), lambda i: (0, i, 0)),
                  pl.BlockSpec((C_in, NB, HC), lambda i: (0, i, 0)),
                  pl.BlockSpec((C_in, HC, h), lambda i: (0, 0, 0)),
                  pl.BlockSpec((1, h), lambda i: (0, 0))],
        out_specs=pl.BlockSpec((C_out, NB, HC), lambda i: (0, i, 0)),
        out_shape=jax.ShapeDtypeStruct((C_out, n, HC), jnp.float32),
    )(x_c, aggr, w_s, b_f.reshape(1, h))


def _final_body(x_ref, a_ref, w_ref, b_ref, bat_ref,
                wl1_ref, bl1_ref, wl2_ref, bl2_ref, o_ref,
                acc_ref, cnt_ref, *, n_steps, G):
    i = pl.program_id(0)
    C_in = x_ref.shape[0]
    acc = b_ref[...]
    for c in range(C_in):
        acc = acc + jnp.dot(x_ref[c] + a_ref[c], w_ref[c],
                            preferred_element_type=jnp.float32)
    y = jnp.maximum(acc, 0.0)
    bat = bat_ref[0]
    gid = lax.broadcasted_iota(jnp.int32, (G, NB), 0)
    P = (bat == gid).astype(jnp.float32)
    part = jnp.dot(P, y, preferred_element_type=jnp.float32)
    pcnt = jnp.sum(P, axis=1, keepdims=True)

    @pl.when(i == 0)
    def _():
        acc_ref[...] = part
        cnt_ref[...] = pcnt

    @pl.when(i > 0)
    def _():
        acc_ref[...] += part
        cnt_ref[...] += pcnt

    @pl.when(i == n_steps - 1)
    def _():
        pooled = acc_ref[...] / jnp.maximum(cnt_ref[...], 1.0)
        z = jnp.maximum(jnp.dot(pooled, wl1_ref[...],
                                preferred_element_type=jnp.float32)
                        + bl1_ref[...], 0.0)
        z = jnp.dot(z, wl2_ref[...],
                    preferred_element_type=jnp.float32) + bl2_ref[...]
        o_ref[...] = 1.0 / (1.0 + jnp.exp(-z))


def _final_layer(x_c, aggr, w_s, b_f, bat2, Wl1, bl1, Wl2, bl2, G):
    C_in, n, _ = x_c.shape
    h = w_s.shape[2]
    n_steps = n // NB
    hh = Wl1.shape[1]
    return pl.pallas_call(
        functools.partial(_final_body, n_steps=n_steps, G=G),
        grid=(n_steps,),
        in_specs=[pl.BlockSpec((C_in, NB, HC), lambda i: (0, i, 0)),
                  pl.BlockSpec((C_in, NB, HC), lambda i: (0, i, 0)),
                  pl.BlockSpec((C_in, HC, h), lambda i: (0, 0, 0)),
                  pl.BlockSpec((1, h), lambda i: (0, 0)),
                  pl.BlockSpec((1, 1, NB), lambda i: (i, 0, 0)),
                  pl.BlockSpec((h, hh), lambda i: (0, 0)),
                  pl.BlockSpec((1, hh), lambda i: (0, 0)),
                  pl.BlockSpec((hh, 1), lambda i: (0, 0)),
                  pl.BlockSpec((1, 1), lambda i: (0, 0))],
        out_specs=pl.BlockSpec((G, 1), lambda i: (0, 0)),
        out_shape=jax.ShapeDtypeStruct((G, 1), jnp.float32),
        scratch_shapes=[pltpu.VMEM((G, h), jnp.float32),
                        pltpu.VMEM((G, 1), jnp.float32)],
    )(x_c, aggr, w_s, b_f.reshape(1, h), bat2,
      Wl1, bl1.reshape(1, hh), Wl2, bl2.reshape(1, 1))


def kernel(x, edge_index, edge_attr, batch,
           We1, be1, Wn1, bn1,
           We2, be2, Wn2, bn2,
           We3, be3, Wn3, bn3,
           g1, bt1, g2, bt2, g3, bt3,
           Wl1, bl1, Wl2, bl2):
    G = 64
    n, dn = x.shape
    E = edge_index.shape[1]

    ept = -(-E // (NS * BLK)) * BLK
    e_pad = ept * NS
    rpt = ((-(-(n + 1) // NS) + 7) // 8) * 8
    n_pad = rpt * NS

    src = edge_index[0].astype(jnp.int32)
    dst = edge_index[1].astype(jnp.int32)
    pad = e_pad - E
    src2 = jnp.concatenate([src, jnp.zeros((pad,), jnp.int32)]).reshape(-1, 128)
    dst2 = jnp.concatenate([dst, jnp.full((pad,), n, jnp.int32)]).reshape(-1, 128)
    ea_p = jnp.concatenate(
        [edge_attr, jnp.zeros((pad, edge_attr.shape[1]), jnp.float32)])

    inv = 1.0 / jnp.sqrt(jnp.float32(1.0 + 1e-5))

    def fold(Wn, bn, g, bt):
        s = g * inv
        return Wn * s[None, :], bn * s + bt

    Wf1, bf1 = fold(Wn1, bn1, g1, bt1)
    Wf2, bf2 = fold(Wn2, bn2, g2, bt2)
    Wf3, bf3 = fold(Wn3, bn3, g3, bt3)

    x_c = x.reshape(n, dn // HC, HC).transpose(1, 0, 2)
    e1c, e2c, e3c = _edge_lin(ea_p, We1, be1, We2, be2, We3, be3)
    zrows = jnp.zeros((rpt, HC), jnp.float32)

    aggr1 = _sc_message(x_c, e1c, src2, dst2, zrows, n_pad)
    h1 = _node_update(x_c, aggr1, Wf1.reshape(dn // HC, HC, -1), bf1)
    aggr2 = _sc_message(h1, e2c, src2, dst2, zrows, n_pad)
    h2 = _node_update(h1, aggr2, Wf2.reshape(-1, HC, Wf2.shape[1]), bf2)
    aggr3 = _sc_message(h2, e3c, src2, dst2, zrows, n_pad)

    bat2 = batch.astype(jnp.int32).reshape(n // NB, 1, NB)
    return _final_layer(h2, aggr3, Wf3.reshape(-1, HC, Wf3.shape[1]), bf3,
                        bat2, Wl1, bl1, Wl2, bl2, G)


# speedup vs baseline: 1.2794x; 1.0467x over previous
import functools

import jax
import jax.numpy as jnp
from jax import lax
from jax.experimental import pallas as pl
from jax.experimental.pallas import tpu as pltpu
from jax.experimental.pallas import tpu_sc as plsc

NC = 2
NS = 16
L = 16

HC = 32
BLK = 128
EB = 2048
NB = 1000


def _sc_message(h_c, e_c, src2, dst2, zrows, n_pad):
    C = h_c.shape[0]
    c_half = C // NC
    e_pad = e_c.shape[1]
    ept = e_pad // NS
    n_blocks = ept // BLK
    rpt = n_pad // NS

    def body(h_hbm, e_hbm, s_hbm, d_hbm, z_hbm, out_hbm, *rest):
        sidx = rest[0:4]
        didx = rest[4:8]
        ebuf = rest[8:10]
        gbuf = rest[10:12]
        aggr_sh = rest[12]
        sem_i = rest[13:17]
        sem_e = rest[17:19]
        sem_g = rest[19:21]
        core = lax.axis_index("c")
        sub = lax.axis_index("s")
        irow0 = sub * n_blocks
        ebase0 = sub * ept
        last = n_blocks - 1

        def fire_idx(k, s4):
            pltpu.async_copy(s_hbm.at[pl.ds(irow0 + k, 1)], sidx[s4],
                             sem_i[s4])
            pltpu.async_copy(d_hbm.at[pl.ds(irow0 + k, 1)], didx[s4],
                             sem_i[s4])

        def wait_idx(s4):
            pltpu.make_async_copy(s_hbm.at[pl.ds(irow0, 1)], sidx[s4],
                                  sem_i[s4]).wait()
            pltpu.make_async_copy(d_hbm.at[pl.ds(irow0, 1)], didx[s4],
                                  sem_i[s4]).wait()

        def fire_eg(c, k, s2, s4):
            pltpu.async_copy(e_hbm.at[c].at[pl.ds(ebase0 + k * BLK, BLK)],
                             ebuf[s2], sem_e[s2])
            pltpu.async_copy(h_hbm.at[c].at[sidx[s4].at[0]], gbuf[s2],
                             sem_g[s2])

        def wait_eg(c, s2):
            pltpu.make_async_copy(e_hbm.at[c].at[pl.ds(ebase0, BLK)],
                                  ebuf[s2], sem_e[s2]).wait()
            pltpu.make_async_copy(h_hbm.at[c].at[sidx[0].at[0]],
                                  gbuf[s2], sem_g[s2]).wait()

        for cc in range(c_half):
            c = core * c_half + cc
            pltpu.sync_copy(z_hbm, aggr_sh.at[pl.ds(sub * rpt, rpt)])
            plsc.subcore_barrier()

            pltpu.sync_copy(s_hbm.at[pl.ds(irow0, 1)], sidx[0])
            pltpu.sync_copy(d_hbm.at[pl.ds(irow0, 1)], didx[0])
            fire_eg(c, 0, 0, 0)
            fire_idx(1, 1)

            @pl.loop(0, n_blocks // 4)
            def _quad(gq):
                for b in range(4):
                    g = gq * 4 + b
                    p2 = b % 2
                    q2 = (b + 1) % 2
                    fire_idx(jnp.minimum(g + 2, last), (b + 2) % 4)
                    wait_idx((b + 1) % 4)
                    fire_eg(c, jnp.minimum(g + 1, last), q2, (b + 1) % 4)
                    wait_eg(c, p2)

                    @pl.loop(0, BLK, unroll=8)
                    def _valu(r):
                        for k in range(HC // L):
                            sl = pl.ds(k * L, L)
                            v = gbuf[p2][r, sl] + ebuf[p2][r, sl]
                            gbuf[p2][r, sl] = jnp.maximum(v, 0.0)

                    pltpu.sync_copy(gbuf[p2], aggr_sh.at[didx[b].at[0]],
                                    add=True)

            wait_idx(1)
            wait_eg(c, 0)

            plsc.subcore_barrier()
            pltpu.sync_copy(aggr_sh.at[pl.ds(sub * rpt, rpt)],
                            out_hbm.at[c].at[pl.ds(sub * rpt, rpt)])
            plsc.subcore_barrier()

    fn = pl.kernel(
        body,
        out_type=jax.ShapeDtypeStruct((C, n_pad, HC), jnp.float32),
        mesh=plsc.VectorSubcoreMesh(core_axis_name="c", subcore_axis_name="s"),
        compiler_params=pltpu.CompilerParams(use_tc_tiling_on_sc=False),
        scratch_types=(
            [pltpu.VMEM((1, 128), jnp.int32)] * 8
            + [pltpu.VMEM((BLK, HC), jnp.float32)] * 4
            + [pltpu.VMEM_SHARED((n_pad, HC), jnp.float32)]
            + [pltpu.SemaphoreType.DMA] * 8
        ),
    )
    return fn(h_c, e_c, src2, dst2, zrows)


def _edge_body(ea_ref, w_ref, b_ref, o_ref):
    ea = ea_ref[...]
    C = o_ref.shape[0]
    for c in range(C):
        y = jnp.dot(ea, w_ref[:, c * HC:(c + 1) * HC],
                    preferred_element_type=jnp.float32)
        o_ref[c] = y + b_ref[0, c * HC:(c + 1) * HC][None, :]


def _edge_lin(ea, We, be, e_pad):
    E, de = ea.shape
    d = We.shape[1]
    grid = (pl.cdiv(E, EB),)
    return pl.pallas_call(
        _edge_body,
        grid=grid,
        in_specs=[pl.BlockSpec((EB, de), lambda i: (i, 0)),
                  pl.BlockSpec((de, d), lambda i: (0, 0)),
                  pl.BlockSpec((1, d), lambda i: (0, 0))],
        out_specs=pl.BlockSpec((d // HC, EB, HC), lambda i: (0, i, 0)),
        out_shape=jax.ShapeDtypeStruct((d // HC, e_pad, HC), jnp.float32),
    )(ea, We, be.reshape(1, d))


def _node_body(x_ref, a_ref, w_ref, b_ref, o_ref):
    C_in = x_ref.shape[0]
    acc = b_ref[...]
    for c in range(C_in):
        acc = acc + jnp.dot(x_ref[c] + a_ref[c], w_ref[c],
                            preferred_element_type=jnp.float32)
    y = jnp.maximum(acc, 0.0)
    C_out = o_ref.shape[0]
    for c in range(C_out):
        o_ref[c] = y[:, c * HC:(c + 1) * HC]


def _node_update(x_c, aggr, w_s, b_f):
    C_in, n, _ = x_c.shape
    h = w_s.shape[2]
    C_out = h // HC
    grid = (n // NB,)
    return pl.pallas_call(
        _node_body,
        grid=grid,
        in_specs=[pl.BlockSpec((C_in, NB, HC), lambda i: (0, i, 0)),
                  pl.BlockSpec((C_in, NB, HC), lambda i: (0, i, 0)),
                  pl.BlockSpec((C_in, HC, h), lambda i: (0, 0, 0)),
                  pl.BlockSpec((1, h), lambda i: (0, 0))],
        out_specs=pl.BlockSpec((C_out, NB, HC), lambda i: (0, i, 0)),
        out_shape=jax.ShapeDtypeStruct((C_out, n, HC), jnp.float32),
    )(x_c, aggr, w_s, b_f.reshape(1, h))


def _final_body(x_ref, a_ref, w_ref, b_ref, bat_ref,
                wl1_ref, bl1_ref, wl2_ref, bl2_ref, o_ref,
                acc_ref, cnt_ref, *, n_steps, G):
    i = pl.program_id(0)
    C_in = x_ref.shape[0]
    acc = b_ref[...]
    for c in range(C_in):
        acc = acc + jnp.dot(x_ref[c] + a_ref[c], w_ref[c],
                            preferred_element_type=jnp.float32)
    y = jnp.maximum(acc, 0.0)
    bat = bat_ref[0]
    gid = lax.broadcasted_iota(jnp.int32, (G, NB), 0)
    P = (bat == gid).astype(jnp.float32)
    part = jnp.dot(P, y, preferred_element_type=jnp.float32)
    pcnt = jnp.sum(P, axis=1, keepdims=True)

    @pl.when(i == 0)
    def _():
        acc_ref[...] = part
        cnt_ref[...] = pcnt

    @pl.when(i > 0)
    def _():
        acc_ref[...] += part
        cnt_ref[...] += pcnt

    @pl.when(i == n_steps - 1)
    def _():
        pooled = acc_ref[...] / jnp.maximum(cnt_ref[...], 1.0)
        z = jnp.maximum(jnp.dot(pooled, wl1_ref[...],
                                preferred_element_type=jnp.float32)
                        + bl1_ref[...], 0.0)
        z = jnp.dot(z, wl2_ref[...],
                    preferred_element_type=jnp.float32) + bl2_ref[...]
        o_ref[...] = 1.0 / (1.0 + jnp.exp(-z))


def _final_layer(x_c, aggr, w_s, b_f, bat2, Wl1, bl1, Wl2, bl2, G):
    C_in, n, _ = x_c.shape
    h = w_s.shape[2]
    n_steps = n // NB
    hh = Wl1.shape[1]
    return pl.pallas_call(
        functools.partial(_final_body, n_steps=n_steps, G=G),
        grid=(n_steps,),
        in_specs=[pl.BlockSpec((C_in, NB, HC), lambda i: (0, i, 0)),
                  pl.BlockSpec((C_in, NB, HC), lambda i: (0, i, 0)),
                  pl.BlockSpec((C_in, HC, h), lambda i: (0, 0, 0)),
                  pl.BlockSpec((1, h), lambda i: (0, 0)),
                  pl.BlockSpec((1, 1, NB), lambda i: (i, 0, 0)),
                  pl.BlockSpec((h, hh), lambda i: (0, 0)),
                  pl.BlockSpec((1, hh), lambda i: (0, 0)),
                  pl.BlockSpec((hh, 1), lambda i: (0, 0)),
                  pl.BlockSpec((1, 1), lambda i: (0, 0))],
        out_specs=pl.BlockSpec((G, 1), lambda i: (0, 0)),
        out_shape=jax.ShapeDtypeStruct((G, 1), jnp.float32),
        scratch_shapes=[pltpu.VMEM((G, h), jnp.float32),
                        pltpu.VMEM((G, 1), jnp.float32)],
    )(x_c, aggr, w_s, b_f.reshape(1, h), bat2,
      Wl1, bl1.reshape(1, hh), Wl2, bl2.reshape(1, 1))


def kernel(x, edge_index, edge_attr, batch,
           We1, be1, Wn1, bn1,
           We2, be2, Wn2, bn2,
           We3, be3, Wn3, bn3,
           g1, bt1, g2, bt2, g3, bt3,
           Wl1, bl1, Wl2, bl2):
    G = 64
    n, dn = x.shape
    E = edge_index.shape[1]

    ept = -(-E // (NS * BLK)) * BLK
    e_pad = ept * NS
    rpt = ((-(-(n + 1) // NS) + 7) // 8) * 8
    n_pad = rpt * NS

    src = edge_index[0].astype(jnp.int32)
    dst = edge_index[1].astype(jnp.int32)
    pad = e_pad - E
    src2 = jnp.concatenate([src, jnp.zeros((pad,), jnp.int32)]).reshape(-1, 128)
    dst2 = jnp.concatenate([dst, jnp.full((pad,), n, jnp.int32)]).reshape(-1, 128)

    inv = 1.0 / jnp.sqrt(jnp.float32(1.0 + 1e-5))

    def fold(Wn, bn, g, bt):
        s = g * inv
        return Wn * s[None, :], bn * s + bt

    Wf1, bf1 = fold(Wn1, bn1, g1, bt1)
    Wf2, bf2 = fold(Wn2, bn2, g2, bt2)
    Wf3, bf3 = fold(Wn3, bn3, g3, bt3)

    x_c = x.reshape(n, dn // HC, HC).transpose(1, 0, 2)
    zrows = jnp.zeros((rpt, HC), jnp.float32)

    e1c = _edge_lin(edge_attr, We1, be1, e_pad)
    aggr1 = _sc_message(x_c, e1c, src2, dst2, zrows, n_pad)
    e2c = _edge_lin(edge_attr, We2, be2, e_pad)
    e3c = _edge_lin(edge_attr, We3, be3, e_pad)
    h1 = _node_update(x_c, aggr1, Wf1.reshape(dn // HC, HC, -1), bf1)
    aggr2 = _sc_message(h1, e2c, src2, dst2, zrows, n_pad)
    h2 = _node_update(h1, aggr2, Wf2.reshape(-1, HC, Wf2.shape[1]), bf2)
    aggr3 = _sc_message(h2, e3c, src2, dst2, zrows, n_pad)

    bat2 = batch.astype(jnp.int32).reshape(n // NB, 1, NB)
    return _final_layer(h2, aggr3, Wf3.reshape(-1, HC, Wf3.shape[1]), bf3,
                        bat2, Wl1, bl1, Wl2, bl2, G)


# speedup vs baseline: 2.8703x; 2.2435x over previous
import functools

import jax
import jax.numpy as jnp
from jax import lax
from jax.experimental import pallas as pl
from jax.experimental.pallas import tpu as pltpu
from jax.experimental.pallas import tpu_sc as plsc

NC = 2
NS = 16
L = 16

HC = 32
BLK = 128
EB4 = 512
NB = 1000


def _sc_message(h_c, e_c, src2, dst2, zrows, n_pad):
    C = h_c.shape[0]
    c_half = C // NC
    e_pad = e_c.shape[1] * 4
    ept = e_pad // NS
    n_blocks = ept // BLK
    rpt = n_pad // NS

    def body(h_hbm, e_hbm, s_hbm, d_hbm, z_hbm, out_hbm, *rest):
        sidx = rest[0:4]
        didx = rest[4:8]
        ebuf = rest[8:10]
        gbuf = rest[10:12]
        aggr_sh = rest[12]
        sem_i = rest[13:17]
        sem_e = rest[17:19]
        sem_g = rest[19:21]
        sem_s = rest[21:23]
        core = lax.axis_index("c")
        sub = lax.axis_index("s")
        irow0 = sub * n_blocks
        erow0 = sub * (ept // 4)
        EBR = BLK // 4
        last = n_blocks - 1

        def fire_idx(k, s4):
            pltpu.async_copy(s_hbm.at[pl.ds(irow0 + k, 1)], sidx[s4],
                             sem_i[s4])
            pltpu.async_copy(d_hbm.at[pl.ds(irow0 + k, 1)], didx[s4],
                             sem_i[s4])

        def wait_idx(s4):
            pltpu.make_async_copy(s_hbm.at[pl.ds(irow0, 1)], sidx[s4],
                                  sem_i[s4]).wait()
            pltpu.make_async_copy(d_hbm.at[pl.ds(irow0, 1)], didx[s4],
                                  sem_i[s4]).wait()

        def fire_eg(c, k, s2, s4):
            pltpu.async_copy(e_hbm.at[c].at[pl.ds(erow0 + k * EBR, EBR)],
                             ebuf[s2], sem_e[s2])
            pltpu.async_copy(h_hbm.at[c].at[sidx[s4].at[0]], gbuf[s2],
                             sem_g[s2])

        def wait_eg(c, s2):
            pltpu.make_async_copy(e_hbm.at[c].at[pl.ds(erow0, EBR)],
                                  ebuf[s2], sem_e[s2]).wait()
            pltpu.make_async_copy(h_hbm.at[c].at[sidx[0].at[0]],
                                  gbuf[s2], sem_g[s2]).wait()

        for cc in range(c_half):
            c = core * c_half + cc
            pltpu.sync_copy(z_hbm, aggr_sh.at[pl.ds(sub * rpt, rpt)])
            plsc.subcore_barrier()

            pltpu.sync_copy(s_hbm.at[pl.ds(irow0, 1)], sidx[0])
            pltpu.sync_copy(d_hbm.at[pl.ds(irow0, 1)], didx[0])
            fire_eg(c, 0, 0, 0)
            fire_idx(1, 1)

            @pl.loop(0, n_blocks // 4)
            def _quad(gq):
                for b in range(4):
                    g = gq * 4 + b
                    p2 = b % 2
                    q2 = (b + 1) % 2
                    fire_idx(jnp.minimum(g + 2, last), (b + 2) % 4)
                    wait_idx((b + 1) % 4)
                    if b == 0:
                        @pl.when(g >= 1)
                        def _():
                            pltpu.make_async_copy(
                                gbuf[q2], aggr_sh.at[didx[0].at[0]],
                                sem_s[q2]).wait()
                    else:
                        pltpu.make_async_copy(
                            gbuf[q2], aggr_sh.at[didx[0].at[0]],
                            sem_s[q2]).wait()
                    fire_eg(c, jnp.minimum(g + 1, last), q2, (b + 1) % 4)
                    wait_eg(c, p2)

                    @pl.loop(0, BLK // 4, unroll=16)
                    def _valu(r):
                        for j in range(4):
                            for k in range(HC // L):
                                gsl = pl.ds(k * L, L)
                                esl = pl.ds(j * HC + k * L, L)
                                v = gbuf[p2][r * 4 + j, gsl] + ebuf[p2][r, esl]
                                gbuf[p2][r * 4 + j, gsl] = jnp.maximum(v, 0.0)

                    pltpu.async_copy(gbuf[p2], aggr_sh.at[didx[b].at[0]],
                                     sem_s[p2], add=True)

            wait_idx(1)
            wait_eg(c, 0)
            pltpu.make_async_copy(gbuf[1], aggr_sh.at[didx[0].at[0]],
                                  sem_s[1]).wait()

            plsc.subcore_barrier()
            pltpu.sync_copy(aggr_sh.at[pl.ds(sub * rpt, rpt)],
                            out_hbm.at[c].at[pl.ds(sub * rpt, rpt)])
            plsc.subcore_barrier()

    fn = pl.kernel(
        body,
        out_type=jax.ShapeDtypeStruct((C, n_pad, HC), jnp.float32),
        mesh=plsc.VectorSubcoreMesh(core_axis_name="c", subcore_axis_name="s"),
        compiler_params=pltpu.CompilerParams(use_tc_tiling_on_sc=False),
        scratch_types=(
            [pltpu.VMEM((1, 128), jnp.int32)] * 8
            + [pltpu.VMEM((BLK // 4, 128), jnp.float32)] * 2
            + [pltpu.VMEM((BLK, HC), jnp.float32)] * 2
            + [pltpu.VMEM_SHARED((n_pad, HC), jnp.float32)]
            + [pltpu.SemaphoreType.DMA] * 10
        ),
    )
    return fn(h_c, e_c, src2, dst2, zrows)


def _edge_body(ea_ref, w_ref, b_ref, o_ref):
    ea = ea_ref[...]
    C = o_ref.shape[0]
    for c in range(C):
        y = jnp.dot(ea, w_ref[c], preferred_element_type=jnp.float32)
        o_ref[c] = y + b_ref[c]


def _edge_lin(ea4, W4, b4, e_pad):
    E4 = ea4.shape[0]
    C = W4.shape[0]
    grid = (pl.cdiv(E4, EB4),)
    return pl.pallas_call(
        _edge_body,
        grid=grid,
        in_specs=[pl.BlockSpec((EB4, 128), lambda i: (i, 0)),
                  pl.BlockSpec((C, 128, 128), lambda i: (0, 0, 0)),
                  pl.BlockSpec((C, 1, 128), lambda i: (0, 0, 0))],
        out_specs=pl.BlockSpec((C, EB4, 128), lambda i: (0, i, 0)),
        out_shape=jax.ShapeDtypeStruct((C, e_pad // 4, 128), jnp.float32),
    )(ea4, W4, b4)


def _node_body(x_ref, a_ref, w_ref, b_ref, o_ref):
    C_in = x_ref.shape[0]
    acc = b_ref[...]
    for c in range(C_in):
        acc = acc + jnp.dot(x_ref[c] + a_ref[c], w_ref[c],
                            preferred_element_type=jnp.float32)
    y = jnp.maximum(acc, 0.0)
    C_out = o_ref.shape[0]
    for c in range(C_out):
        o_ref[c] = y[:, c * HC:(c + 1) * HC]


def _node_update(x_c, aggr, w_s, b_f):
    C_in, n, _ = x_c.shape
    h = w_s.shape[2]
    C_out = h // HC
    grid = (n // NB,)
    return pl.pallas_call(
        _node_body,
        grid=grid,
        in_specs=[pl.BlockSpec((C_in, NB, HC), lambda i: (0, i, 0)),
                  pl.BlockSpec((C_in, NB, HC), lambda i: (0, i, 0)),
                  pl.BlockSpec((C_in, HC, h), lambda i: (0, 0, 0)),
                  pl.BlockSpec((1, h), lambda i: (0, 0))],
        out_specs=pl.BlockSpec((C_out, NB, HC), lambda i: (0, i, 0)),
        out_shape=jax.ShapeDtypeStruct((C_out, n, HC), jnp.float32),
    )(x_c, aggr, w_s, b_f.reshape(1, h))


def _final_body(x_ref, a_ref, w_ref, b_ref, bat_ref,
                wl1_ref, bl1_ref, wl2_ref, bl2_ref, o_ref,
                acc_ref, cnt_ref, *, n_steps, G):
    i = pl.program_id(0)
    C_in = x_ref.shape[0]
    acc = b_ref[...]
    for c in range(C_in):
        acc = acc + jnp.dot(x_ref[c] + a_ref[c], w_ref[c],
                            preferred_element_type=jnp.float32)
    y = jnp.maximum(acc, 0.0)
    bat = bat_ref[0]
    gid = lax.broadcasted_iota(jnp.int32, (G, NB), 0)
    P = (bat == gid).astype(jnp.float32)
    part = jnp.dot(P, y, preferred_element_type=jnp.float32)
    pcnt = jnp.sum(P, axis=1, keepdims=True)

    @pl.when(i == 0)
    def _():
        acc_ref[...] = part
        cnt_ref[...] = pcnt

    @pl.when(i > 0)
    def _():
        acc_ref[...] += part
        cnt_ref[...] += pcnt

    @pl.when(i == n_steps - 1)
    def _():
        pooled = acc_ref[...] / jnp.maximum(cnt_ref[...], 1.0)
        z = jnp.maximum(jnp.dot(pooled, wl1_ref[...],
                                preferred_element_type=jnp.float32)
                        + bl1_ref[...], 0.0)
        z = jnp.dot(z, wl2_ref[...],
                    preferred_element_type=jnp.float32) + bl2_ref[...]
        o_ref[...] = 1.0 / (1.0 + jnp.exp(-z))


def _final_layer(x_c, aggr, w_s, b_f, bat2, Wl1, bl1, Wl2, bl2, G):
    C_in, n, _ = x_c.shape
    h = w_s.shape[2]
    n_steps = n // NB
    hh = Wl1.shape[1]
    return pl.pallas_call(
        functools.partial(_final_body, n_steps=n_steps, G=G),
        grid=(n_steps,),
        in_specs=[pl.BlockSpec((C_in, NB, HC), lambda i: (0, i, 0)),
                  pl.BlockSpec((C_in, NB, HC), lambda i: (0, i, 0)),
                  pl.BlockSpec((C_in, HC, h), lambda i: (0, 0, 0)),
                  pl.BlockSpec((1, h), lambda i: (0, 0)),
                  pl.BlockSpec((1, 1, NB), lambda i: (i, 0, 0)),
                  pl.BlockSpec((h, hh), lambda i: (0, 0)),
                  pl.BlockSpec((1, hh), lambda i: (0, 0)),
                  pl.BlockSpec((hh, 1), lambda i: (0, 0)),
                  pl.BlockSpec((1, 1), lambda i: (0, 0))],
        out_specs=pl.BlockSpec((G, 1), lambda i: (0, 0)),
        out_shape=jax.ShapeDtypeStruct((G, 1), jnp.float32),
        scratch_shapes=[pltpu.VMEM((G, h), jnp.float32),
                        pltpu.VMEM((G, 1), jnp.float32)],
    )(x_c, aggr, w_s, b_f.reshape(1, h), bat2,
      Wl1, bl1.reshape(1, hh), Wl2, bl2.reshape(1, 1))


def kernel(x, edge_index, edge_attr, batch,
           We1, be1, Wn1, bn1,
           We2, be2, Wn2, bn2,
           We3, be3, Wn3, bn3,
           g1, bt1, g2, bt2, g3, bt3,
           Wl1, bl1, Wl2, bl2):
    G = 64
    n, dn = x.shape
    E = edge_index.shape[1]

    ept = -(-E // (NS * BLK)) * BLK
    e_pad = ept * NS
    rpt = ((-(-(n + 1) // NS) + 7) // 8) * 8
    n_pad = rpt * NS

    src = edge_index[0].astype(jnp.int32)
    dst = edge_index[1].astype(jnp.int32)
    pad = e_pad - E
    src2 = jnp.concatenate([src, jnp.zeros((pad,), jnp.int32)]).reshape(-1, 128)
    dst2 = jnp.concatenate([dst, jnp.full((pad,), n, jnp.int32)]).reshape(-1, 128)

    inv = 1.0 / jnp.sqrt(jnp.float32(1.0 + 1e-5))

    def fold(Wn, bn, g, bt):
        s = g * inv
        return Wn * s[None, :], bn * s + bt

    Wf1, bf1 = fold(Wn1, bn1, g1, bt1)
    Wf2, bf2 = fold(Wn2, bn2, g2, bt2)
    Wf3, bf3 = fold(Wn3, bn3, g3, bt3)

    x_c = x.reshape(n, dn // HC, HC).transpose(1, 0, 2)
    zrows = jnp.zeros((rpt, HC), jnp.float32)

    ea4 = edge_attr.reshape(E // 4, 4 * edge_attr.shape[1])
    eye4 = jnp.eye(4, dtype=jnp.float32)

    def pack_w(We, be):
        C = We.shape[1] // HC
        W4 = jnp.stack([jnp.kron(eye4, We[:, c * HC:(c + 1) * HC])
                        for c in range(C)])
        b4 = jnp.stack([jnp.tile(be[c * HC:(c + 1) * HC], 4).reshape(1, 128)
                        for c in range(C)])
        return W4, b4

    W41, b41 = pack_w(We1, be1)
    W42, b42 = pack_w(We2, be2)
    W43, b43 = pack_w(We3, be3)

    e1c = _edge_lin(ea4, W41, b41, e_pad)
    aggr1 = _sc_message(x_c, e1c, src2, dst2, zrows, n_pad)
    e2c = _edge_lin(ea4, W42, b42, e_pad)
    e3c = _edge_lin(ea4, W43, b43, e_pad)
    h1 = _node_update(x_c, aggr1, Wf1.reshape(dn // HC, HC, -1), bf1)
    aggr2 = _sc_message(h1, e2c, src2, dst2, zrows, n_pad)
    h2 = _node_update(h1, aggr2, Wf2.reshape(-1, HC, Wf2.shape[1]), bf2)
    aggr3 = _sc_message(h2, e3c, src2, dst2, zrows, n_pad)

    bat2 = batch.astype(jnp.int32).reshape(n // NB, 1, NB)
    return _final_layer(h2, aggr3, Wf3.reshape(-1, HC, Wf3.shape[1]), bf3,
                        bat2, Wl1, bl1, Wl2, bl2, G)
